# baseline (speedup 1.0000x reference)
import functools

import jax
import jax.numpy as jnp
from jax import lax
from jax.experimental import pallas as pl
from jax.experimental.pallas import tpu as pltpu
from jax.experimental.pallas import tpu_sc as plsc

R = 8
NVOX = R * R * R
LANES = 16
NC, NS = 2, 16
CLS = 40
CPAD = 48

CHUNK = 16384
U1 = 8
U2 = 8
NSLOT = 4


def _body(x_hbm, wt_hbm, bias_hbm, out_hbm,
          b0, b1, b2, b3, b4, b5, hist, wtbuf,
          s0_, s1_, s2_, s3_, s4_, s5_, s6_):
    _, B, N = x_hbm.shape
    bid = lax.axis_index("s") * NC + lax.axis_index("c")

    zeros16 = jnp.zeros((LANES,), jnp.float32)
    ones16 = jnp.ones((LANES,), jnp.float32)
    lane_off = lax.iota(jnp.int32, LANES)
    n_chunks = N // CHUNK
    bufs = (b0, b1, b2, b3, b4, b5)
    sems = (s0_, s1_, s2_, s3_, s4_, s5_)

    def start(d, c, slot):
        return pltpu.async_copy(
            x_hbm.at[d, bid, pl.ds(c * CHUNK, CHUNK)], bufs[slot], sems[slot])

    def start3(c, phase):
        return [start(d, c, 3 * phase + d) for d in range(3)]

    pending3 = start3(0, 0)
    wt_copy = pltpu.async_copy(wt_hbm, wtbuf, s6_)

    def zero_body(i, _):
        hist[pl.ds(i * LANES, LANES)] = zeros16
        return 0

    lax.fori_loop(0, (LANES * NVOX) // LANES, zero_body, 0)

    n1_iters = CHUNK // (LANES * U1)
    big = jnp.full((LANES,), jnp.inf, jnp.float32)
    carry = (big,) * (3 * NSLOT) + (-big,) * (3 * NSLOT)
    for c in range(n_chunks):
        phase = c % 2
        if c + 1 < n_chunks:
            nxt3 = start3(c + 1, 1 - phase)
        else:
            nxt3 = start3(0, 1 - phase)
        for h in pending3:
            h.wait()
        bx, by, bz = bufs[3 * phase], bufs[3 * phase + 1], bufs[3 * phase + 2]

        def p1_iter(it, carry, bx=bx, by=by, bz=bz):
            c_ = list(carry)
            for k in range(U1):
                o = (it * U1 + k) * LANES
                s = k % NSLOT
                for d, bd in enumerate((bx, by, bz)):
                    v = bd[pl.ds(o, LANES)]
                    c_[3 * s + d] = jnp.minimum(c_[3 * s + d], v)
                    c_[3 * NSLOT + 3 * s + d] = jnp.maximum(
                        c_[3 * NSLOT + 3 * s + d], v)
            return tuple(c_)

        carry = lax.fori_loop(0, n1_iters, p1_iter, carry)
        pending3 = nxt3

    mns, scs = [], []
    for d in range(3):
        mn, mx = carry[d], carry[3 * NSLOT + d]
        for s in range(1, NSLOT):
            mn = jnp.minimum(mn, carry[3 * s + d])
            mx = jnp.maximum(mx, carry[3 * NSLOT + 3 * s + d])
        mn_s = jnp.broadcast_to(jnp.min(mn), (LANES,))
        mx_s = jnp.broadcast_to(jnp.max(mx), (LANES,))
        width = jnp.where(mx_s > mn_s, mx_s - mn_s,
                          jnp.full((LANES,), 1.0, jnp.float32))
        mns.append(mn_s)
        scs.append(jnp.full((LANES,), float(R), jnp.float32) / width)
    mn_0, mn_1, mn_2 = mns
    sc_0, sc_1, sc_2 = scs

    n_groups = CHUNK // LANES
    for c in range(n_chunks):
        phase = c % 2
        nxt3 = start3(c + 1, 1 - phase) if c + 1 < n_chunks else None
        for h in pending3:
            h.wait()
        bx, by, bz = bufs[3 * phase], bufs[3 * phase + 1], bufs[3 * phase + 2]

        @plsc.parallel_loop(0, n_groups, unroll=U2)
        def p2_group(g, bx=bx, by=by, bz=bz):
            o = g * LANES
            v0 = bx[pl.ds(o, LANES)]
            v1 = by[pl.ds(o, LANES)]
            v2 = bz[pl.ds(o, LANES)]
            i0 = jnp.minimum(((v0 - mn_0) * sc_0).astype(jnp.int32), R - 1)
            i1 = jnp.minimum(((v1 - mn_1) * sc_1).astype(jnp.int32), R - 1)
            i2 = jnp.minimum(((v2 - mn_2) * sc_2).astype(jnp.int32), R - 1)
            vox = ((i0 * R + i1) * R + i2) * LANES + lane_off
            plsc.addupdate_scatter(hist, [vox], ones16)

        pending3 = nxt3

    inv_n = jnp.float32(1.0 / N)
    lane16 = lax.iota(jnp.int32, LANES) * LANES
    pltpu.sync_copy(bias_hbm, b0.at[pl.ds(0, CPAD)])
    wt_copy.wait()

    def gemm_body(g, carry):
        o0, o1, o2 = carry
        base = g * (LANES * LANES) + lane16
        acc = plsc.load_gather(hist, [base])
        for j in range(1, LANES):
            acc = acc + plsc.load_gather(hist, [base + j])
        acc = acc * inv_n
        for j in range(LANES):
            s = jnp.broadcast_to(acc[j], (LANES,))
            ro = (g * LANES + j) * CLS
            o0 = o0 + s * wtbuf[pl.ds(ro, LANES)]
            o1 = o1 + s * wtbuf[pl.ds(ro + LANES, LANES)]
            o2 = o2 + s * wtbuf[pl.ds(ro + 2 * LANES, LANES)]
        return (o0, o1, o2)

    o0, o1, o2 = lax.fori_loop(0, NVOX // LANES, gemm_body,
                               (zeros16, zeros16, zeros16))
    o0 = o0 + b0[pl.ds(0, LANES)]
    o1 = o1 + b0[pl.ds(LANES, LANES)]
    o2 = o2 + b0[pl.ds(2 * LANES, LANES)]
    hist[pl.ds(0, LANES)] = o0
    hist[pl.ds(LANES, LANES)] = o1
    hist[pl.ds(2 * LANES, LANES)] = o2
    pltpu.sync_copy(hist.at[pl.ds(0, CLS)], out_hbm.at[pl.ds(bid * CLS, CLS)])


def _sc_fused(xt, wt, bias):
    _, B, N = xt.shape
    mesh = plsc.VectorSubcoreMesh(core_axis_name="c", subcore_axis_name="s",
                                  num_cores=NC, num_subcores=NS)
    return pl.kernel(
        _body,
        out_type=jax.ShapeDtypeStruct((B * CLS,), jnp.float32),
        mesh=mesh,
        compiler_params=pltpu.CompilerParams(needs_layout_passes=False),
        scratch_types=[
            pltpu.VMEM((CHUNK,), jnp.float32),
            pltpu.VMEM((CHUNK,), jnp.float32),
            pltpu.VMEM((CHUNK,), jnp.float32),
            pltpu.VMEM((CHUNK,), jnp.float32),
            pltpu.VMEM((CHUNK,), jnp.float32),
            pltpu.VMEM((CHUNK,), jnp.float32),
            pltpu.VMEM((LANES * NVOX,), jnp.float32),
            pltpu.VMEM(((NVOX + 1) * CLS,), jnp.float32),
            pltpu.SemaphoreType.DMA,
            pltpu.SemaphoreType.DMA,
            pltpu.SemaphoreType.DMA,
            pltpu.SemaphoreType.DMA,
            pltpu.SemaphoreType.DMA,
            pltpu.SemaphoreType.DMA,
            pltpu.SemaphoreType.DMA,
        ],
    )(xt, wt, bias)


@jax.jit
def kernel(x, W, b):
    xt = jnp.transpose(x, (2, 0, 1))
    wt = jnp.pad(W.T, ((0, 1), (0, 0))).reshape(-1)
    bias = jnp.pad(b, (0, CPAD - CLS))
    return _sc_fused(xt, wt, bias).reshape(x.shape[0], CLS)


# speedup vs baseline: 1.1313x; 1.1313x over previous
import functools

import jax
import jax.numpy as jnp
from jax import lax
from jax.experimental import pallas as pl
from jax.experimental.pallas import tpu as pltpu
from jax.experimental.pallas import tpu_sc as plsc

R = 8
NVOX = R * R * R
LANES = 16
NC, NS = 2, 16

CHUNK = 16384
U1 = 8
U2 = 8
NSLOT = 4


def _hist_body(x_hbm, counts_hbm,
               b0, b1, b2, b3, b4, b5, hist, cnt,
               s0_, s1_, s2_, s3_, s4_, s5_):
    _, B, N = x_hbm.shape
    bid = lax.axis_index("s") * NC + lax.axis_index("c")

    zeros16 = jnp.zeros((LANES,), jnp.float32)
    ones16 = jnp.ones((LANES,), jnp.float32)
    lane_off = lax.iota(jnp.int32, LANES)
    n_chunks = N // CHUNK
    bufs = (b0, b1, b2, b3, b4, b5)
    sems = (s0_, s1_, s2_, s3_, s4_, s5_)

    def start(d, c, slot):
        return pltpu.async_copy(
            x_hbm.at[d, bid, pl.ds(c * CHUNK, CHUNK)], bufs[slot], sems[slot])

    def start3(c, phase):
        return [start(d, c, 3 * phase + d) for d in range(3)]

    pending3 = start3(0, 0)

    def zero_body(i, _):
        hist[pl.ds(i * LANES, LANES)] = zeros16
        return 0

    lax.fori_loop(0, (LANES * NVOX) // LANES, zero_body, 0)

    n1_iters = CHUNK // (LANES * U1)
    big = jnp.full((LANES,), jnp.inf, jnp.float32)
    carry = (big,) * (3 * NSLOT) + (-big,) * (3 * NSLOT)
    for c in range(n_chunks):
        phase = c % 2
        if c + 1 < n_chunks:
            nxt3 = start3(c + 1, 1 - phase)
        else:
            nxt3 = start3(0, 1 - phase)
        for h in pending3:
            h.wait()
        bx, by, bz = bufs[3 * phase], bufs[3 * phase + 1], bufs[3 * phase + 2]

        def p1_iter(it, carry, bx=bx, by=by, bz=bz):
            c_ = list(carry)
            for k in range(U1):
                o = (it * U1 + k) * LANES
                s = k % NSLOT
                for d, bd in enumerate((bx, by, bz)):
                    v = bd[pl.ds(o, LANES)]
                    c_[3 * s + d] = jnp.minimum(c_[3 * s + d], v)
                    c_[3 * NSLOT + 3 * s + d] = jnp.maximum(
                        c_[3 * NSLOT + 3 * s + d], v)
            return tuple(c_)

        carry = lax.fori_loop(0, n1_iters, p1_iter, carry)
        pending3 = nxt3

    mns, scs = [], []
    for d in range(3):
        mn, mx = carry[d], carry[3 * NSLOT + d]
        for s in range(1, NSLOT):
            mn = jnp.minimum(mn, carry[3 * s + d])
            mx = jnp.maximum(mx, carry[3 * NSLOT + 3 * s + d])
        mn_s = jnp.broadcast_to(jnp.min(mn), (LANES,))
        mx_s = jnp.broadcast_to(jnp.max(mx), (LANES,))
        width = jnp.where(mx_s > mn_s, mx_s - mn_s,
                          jnp.full((LANES,), 1.0, jnp.float32))
        mns.append(mn_s)
        scs.append(jnp.full((LANES,), float(R), jnp.float32) / width)
    mn_0, mn_1, mn_2 = mns
    sc_0, sc_1, sc_2 = scs

    magic = jnp.full((LANES,), 8388607.5, jnp.float32)
    k_0 = magic - mn_0 * sc_0
    k_1 = magic - mn_1 * sc_1
    k_2 = magic - mn_2 * sc_2
    lim = jnp.full((LANES,), 8388615.0, jnp.float32)
    lo = jnp.full((LANES,), 8388608.0, jnp.float32)
    seven = jnp.full((LANES,), 7, jnp.int32)

    n_groups = CHUNK // LANES
    for c in range(n_chunks):
        phase = c % 2
        nxt3 = start3(c + 1, 1 - phase) if c + 1 < n_chunks else None
        for h in pending3:
            h.wait()
        bx, by, bz = bufs[3 * phase], bufs[3 * phase + 1], bufs[3 * phase + 2]

        @plsc.parallel_loop(0, n_groups, unroll=U2)
        def p2_group(g, bx=bx, by=by, bz=bz):
            o = g * LANES
            v0 = bx[pl.ds(o, LANES)]
            v1 = by[pl.ds(o, LANES)]
            v2 = bz[pl.ds(o, LANES)]
            y0 = jnp.maximum(jnp.minimum(v0 * sc_0 + k_0, lim), lo)
            y1 = jnp.maximum(jnp.minimum(v1 * sc_1 + k_1, lim), lo)
            y2 = jnp.maximum(jnp.minimum(v2 * sc_2 + k_2, lim), lo)
            i0 = plsc.bitcast(y0, jnp.int32) & seven
            i1 = plsc.bitcast(y1, jnp.int32) & seven
            i2 = plsc.bitcast(y2, jnp.int32) & seven
            vox = ((i0 * R + i1) * R + i2) * LANES + lane_off
            plsc.addupdate_scatter(hist, [vox], ones16)

        pending3 = nxt3

    inv_n = jnp.float32(1.0 / N)
    lane16 = lax.iota(jnp.int32, LANES) * LANES

    def merge_body(g, _):
        base = g * (LANES * LANES) + lane16
        acc = plsc.load_gather(hist, [base])
        for j in range(1, LANES):
            acc = acc + plsc.load_gather(hist, [base + j])
        cnt[pl.ds(g * LANES, LANES)] = acc * inv_n
        return 0

    lax.fori_loop(0, NVOX // LANES, merge_body, 0)
    pltpu.sync_copy(cnt, counts_hbm.at[bid])


def _sc_counts(xt):
    _, B, N = xt.shape
    mesh = plsc.VectorSubcoreMesh(core_axis_name="c", subcore_axis_name="s",
                                  num_cores=NC, num_subcores=NS)
    return pl.kernel(
        _hist_body,
        out_type=jax.ShapeDtypeStruct((B, NVOX), jnp.float32),
        mesh=mesh,
        compiler_params=pltpu.CompilerParams(needs_layout_passes=False),
        scratch_types=[
            pltpu.VMEM((CHUNK,), jnp.float32),
            pltpu.VMEM((CHUNK,), jnp.float32),
            pltpu.VMEM((CHUNK,), jnp.float32),
            pltpu.VMEM((CHUNK,), jnp.float32),
            pltpu.VMEM((CHUNK,), jnp.float32),
            pltpu.VMEM((CHUNK,), jnp.float32),
            pltpu.VMEM((LANES * NVOX,), jnp.float32),
            pltpu.VMEM((NVOX,), jnp.float32),
            pltpu.SemaphoreType.DMA,
            pltpu.SemaphoreType.DMA,
            pltpu.SemaphoreType.DMA,
            pltpu.SemaphoreType.DMA,
            pltpu.SemaphoreType.DMA,
            pltpu.SemaphoreType.DMA,
        ],
    )(xt)


def _gemm_body(c_ref, w_ref, b_ref, o_ref):
    o_ref[...] = lax.dot_general(
        c_ref[...], w_ref[...], (((1,), (1,)), ((), ())),
        preferred_element_type=jnp.float32) + b_ref[...]


def _tc_gemm(counts, W, b):
    B = counts.shape[0]
    C = W.shape[0]
    return pl.pallas_call(
        _gemm_body,
        out_shape=jax.ShapeDtypeStruct((B, C), jnp.float32),
    )(counts, W, b.reshape(1, C))


@jax.jit
def kernel(x, W, b):
    xt = jnp.transpose(x, (2, 0, 1))
    counts = _sc_counts(xt)
    return _tc_gemm(counts, W, b)
